# baseline (speedup 1.0000x reference)
import jax
import jax.numpy as jnp
from jax import lax
from jax.experimental import pallas as pl
from jax.experimental.pallas import tpu as pltpu
from jax.experimental.pallas import tpu_sc as plsc

N = 10000
D = 128
E = 320000
NC = 2
NS = 16
NW = NC * NS
CHUNK = 128
CPW = 80
EPAD = NW * CPW * CHUNK
ECH = EPAD // CHUNK
IBLK = 16
NPAD = 10112
RPT = NPAD // NS
XPT = NPAD // NS
DEGW = 128

ROWS = 2000
GRID = N // ROWS

_MESH = dict(core_axis_name="c", subcore_axis_name="s",
             num_cores=NC, num_subcores=NS)


def _deg_body(dst_hbm, ones_hbm, z_hbm, out_hbm, didx, onesv, acc, sem):
    c = lax.axis_index("c")
    s = lax.axis_index("s")
    w = s * NC + c
    pltpu.sync_copy(ones_hbm, onesv)
    pltpu.sync_copy(z_hbm.at[pl.ds(s * RPT, RPT)], acc.at[pl.ds(s * RPT, RPT)])
    plsc.subcore_barrier()

    def step(i, carry):
        off = (w * CPW + i) * CHUNK
        pltpu.sync_copy(dst_hbm.at[pl.ds(off, CHUNK)], didx)
        pltpu.sync_copy(onesv, acc.at[didx], add=True)
        return carry

    lax.fori_loop(0, CPW, step, 0)
    plsc.subcore_barrier()
    pltpu.sync_copy(acc.at[pl.ds(s * XPT, XPT)],
                    out_hbm.at[c, pl.ds(s * XPT, XPT)])


def _deg_call(dstp, ones, zeros):
    f = pl.kernel(
        _deg_body,
        out_type=jax.ShapeDtypeStruct((NC, NPAD, DEGW), jnp.float32),
        mesh=plsc.VectorSubcoreMesh(**_MESH),
        scratch_types=[
            pltpu.VMEM((CHUNK,), jnp.int32),
            pltpu.VMEM((CHUNK, DEGW), jnp.float32),
            pltpu.VMEM_SHARED((NPAD, DEGW), jnp.float32),
            pltpu.SemaphoreType.DMA,
        ],
    )
    return f(dstp, ones, zeros)


def _agg_body(y_hbm, src_hbm, dst_hbm, z_hbm, out_hbm,
              sidx_a, didx_a, rows_a, sidx_b, didx_b, rows_b, acc,
              sem_a, sem_b):
    c = lax.axis_index("c")
    s = lax.axis_index("s")
    w = s * NC + c
    base = w * CPW * CHUNK
    pltpu.sync_copy(z_hbm.at[pl.ds(s * RPT, RPT)], acc.at[pl.ds(s * RPT, RPT)])
    plsc.subcore_barrier()

    def step(i, carry):
        off = base + i * CHUNK
        pltpu.sync_copy(src_hbm.at[pl.ds(off, CHUNK)], sidx_a)
        pltpu.sync_copy(dst_hbm.at[pl.ds(off, CHUNK)], didx_a)
        pltpu.async_copy(y_hbm.at[sidx_a], rows_a, sem_a).wait()
        pltpu.sync_copy(rows_a, acc.at[didx_a], add=True)
        return carry

    lax.fori_loop(0, CPW, step, 0)
    plsc.subcore_barrier()
    pltpu.sync_copy(acc.at[pl.ds(s * XPT, XPT)],
                    out_hbm.at[c, pl.ds(s * XPT, XPT)])


def _agg_call(y, srcp, dstp, zeros):
    f = pl.kernel(
        _agg_body,
        out_type=jax.ShapeDtypeStruct((NC, NPAD, D), jnp.float32),
        mesh=plsc.VectorSubcoreMesh(**_MESH),
        scratch_types=[
            pltpu.VMEM((CHUNK,), jnp.int32),
            pltpu.VMEM((CHUNK,), jnp.int32),
            pltpu.VMEM((CHUNK, D), jnp.float32),
            pltpu.VMEM((CHUNK,), jnp.int32),
            pltpu.VMEM((CHUNK,), jnp.int32),
            pltpu.VMEM((CHUNK, D), jnp.float32),
            pltpu.VMEM_SHARED((NPAD, D), jnp.float32),
            pltpu.SemaphoreType.DMA,
            pltpu.SemaphoreType.DMA,
        ],
    )
    return f(y, srcp, dstp, zeros)


def _dinv(deg_ref):
    dsum = deg_ref[0] + deg_ref[1]
    return lax.rsqrt(dsum[:, 0:1] + 1.0)


def _prep_body(deg_ref, x_ref, w_ref, t_ref, y_ref):
    dinv = _dinv(deg_ref)
    t = jnp.dot(x_ref[...], w_ref[...], preferred_element_type=jnp.float32)
    t_ref[...] = t
    y_ref[...] = dinv * t


def _mid_body(deg_ref, agg_ref, t_ref, b_ref, w_ref, tn_ref, yn_ref):
    dinv = _dinv(deg_ref)
    a = agg_ref[0] + agg_ref[1]
    h = jnp.maximum(dinv * a + (dinv * dinv) * t_ref[...] + b_ref[...], 0.0)
    t = jnp.dot(h, w_ref[...], preferred_element_type=jnp.float32)
    tn_ref[...] = t
    yn_ref[...] = dinv * t


def _final_body(deg_ref, agg_ref, t_ref, b_ref, out_ref):
    dinv = _dinv(deg_ref)
    a = agg_ref[0] + agg_ref[1]
    out_ref[...] = dinv * a + (dinv * dinv) * t_ref[...] + b_ref[...]


_DEG_SPEC = pl.BlockSpec((NC, ROWS, DEGW), lambda i: (0, i, 0))
_AGG_SPEC = pl.BlockSpec((NC, ROWS, D), lambda i: (0, i, 0))
_ROW_SPEC = pl.BlockSpec((ROWS, D), lambda i: (i, 0))
_W_SPEC = pl.BlockSpec((D, D), lambda i: (0, 0))
_B_SPEC = pl.BlockSpec((1, D), lambda i: (0, 0))
_ROW_OUT = jax.ShapeDtypeStruct((N, D), jnp.float32)


def _prep_call(deg, x, W0):
    return pl.pallas_call(
        _prep_body,
        grid=(GRID,),
        in_specs=[_DEG_SPEC, _ROW_SPEC, _W_SPEC],
        out_specs=[_ROW_SPEC, _ROW_SPEC],
        out_shape=[_ROW_OUT, _ROW_OUT],
    )(deg, x, W0)


def _mid_call(deg, agg, t, b, W):
    return pl.pallas_call(
        _mid_body,
        grid=(GRID,),
        in_specs=[_DEG_SPEC, _AGG_SPEC, _ROW_SPEC, _B_SPEC, _W_SPEC],
        out_specs=[_ROW_SPEC, _ROW_SPEC],
        out_shape=[_ROW_OUT, _ROW_OUT],
    )(deg, agg, t, b, W)


def _final_call(deg, agg, t, b):
    return pl.pallas_call(
        _final_body,
        grid=(GRID,),
        in_specs=[_DEG_SPEC, _AGG_SPEC, _ROW_SPEC, _B_SPEC],
        out_specs=_ROW_SPEC,
        out_shape=_ROW_OUT,
    )(deg, agg, t, b)


def kernel(x, edge_index, W0, b0, W1, b1, Wf, bf):
    src = edge_index[0].astype(jnp.int32)
    dst = edge_index[1].astype(jnp.int32)
    pad = EPAD - E
    srcp = jnp.concatenate([src, jnp.zeros((pad,), jnp.int32)])
    dstp = jnp.concatenate([dst, jnp.full((pad,), N, jnp.int32)])
    zeros = jnp.zeros((NPAD, D), jnp.float32)
    ones = jnp.ones((CHUNK, DEGW), jnp.float32)
    b0r = b0.reshape(1, D)
    b1r = b1.reshape(1, D)
    bfr = bf.reshape(1, D)

    deg = _deg_call(dstp, ones, zeros)
    t0, y0 = _prep_call(deg, x, W0)
    agg0 = _agg_call(y0, srcp, dstp, zeros)
    t1, y1 = _mid_call(deg, agg0, t0, b0r, W1)
    agg1 = _agg_call(y1, srcp, dstp, zeros)
    t2, y2 = _mid_call(deg, agg1, t1, b1r, Wf)
    agg2 = _agg_call(y2, srcp, dstp, zeros)
    return _final_call(deg, agg2, t2, bfr)


# speedup vs baseline: 2.0340x; 2.0340x over previous
import jax
import jax.numpy as jnp
from jax import lax
from jax.experimental import pallas as pl
from jax.experimental.pallas import tpu as pltpu
from jax.experimental.pallas import tpu_sc as plsc

N = 10000
D = 128
E = 320000
NC = 2
NS = 16
NW = NC * NS
CHUNK = 128
CPW = 80
EPAD = NW * CPW * CHUNK
ECH = EPAD // CHUNK
IBLK = 16
NPAD = 10112
RPT = NPAD // NS
XPT = NPAD // NS
DEGW = 128

ROWS = 2000
GRID = N // ROWS

_MESH = dict(core_axis_name="c", subcore_axis_name="s",
             num_cores=NC, num_subcores=NS)


def _deg_body(dst_hbm, ones_hbm, z_hbm, out_hbm, didx, onesv, acc, sem):
    c = lax.axis_index("c")
    s = lax.axis_index("s")
    w = s * NC + c
    pltpu.sync_copy(ones_hbm, onesv)
    pltpu.sync_copy(z_hbm.at[pl.ds(s * RPT, RPT)], acc.at[pl.ds(s * RPT, RPT)])
    plsc.subcore_barrier()

    def step(i, carry):
        off = (w * CPW + i) * CHUNK
        pltpu.sync_copy(dst_hbm.at[pl.ds(off, CHUNK)], didx)
        pltpu.sync_copy(onesv, acc.at[didx], add=True)
        return carry

    lax.fori_loop(0, CPW, step, 0)
    plsc.subcore_barrier()
    pltpu.sync_copy(acc.at[pl.ds(s * XPT, XPT)],
                    out_hbm.at[c, pl.ds(s * XPT, XPT)])


def _deg_call(dstp, ones, zeros):
    f = pl.kernel(
        _deg_body,
        out_type=jax.ShapeDtypeStruct((NC, NPAD, DEGW), jnp.float32),
        mesh=plsc.VectorSubcoreMesh(**_MESH),
        scratch_types=[
            pltpu.VMEM((CHUNK,), jnp.int32),
            pltpu.VMEM((CHUNK, DEGW), jnp.float32),
            pltpu.VMEM_SHARED((NPAD, DEGW), jnp.float32),
            pltpu.SemaphoreType.DMA,
        ],
    )
    return f(dstp, ones, zeros)


def _agg_body(y_hbm, src_hbm, dst_hbm, z_hbm, out_hbm,
              sidx_a, didx_a, rows_a, sidx_b, didx_b, rows_b, acc,
              sem_a, sem_b):
    c = lax.axis_index("c")
    s = lax.axis_index("s")
    w = s * NC + c
    base = w * CPW * CHUNK
    pltpu.sync_copy(z_hbm.at[pl.ds(s * RPT, RPT)], acc.at[pl.ds(s * RPT, RPT)])
    plsc.subcore_barrier()

    def step(i, carry):
        off = base + i * CHUNK
        pltpu.sync_copy(src_hbm.at[pl.ds(off, CHUNK)], sidx_a)
        pltpu.sync_copy(dst_hbm.at[pl.ds(off, CHUNK)], didx_a)
        pltpu.async_copy(y_hbm.at[sidx_a], rows_a, sem_a).wait()
        pltpu.sync_copy(rows_a, acc.at[didx_a], add=True)
        return carry

    lax.fori_loop(0, CPW, step, 0)
    plsc.subcore_barrier()
    pltpu.sync_copy(acc.at[pl.ds(s * XPT, XPT)],
                    out_hbm.at[c, pl.ds(s * XPT, XPT)])


def _agg_call(y, srcp, dstp, zeros):
    f = pl.kernel(
        _agg_body,
        out_type=jax.ShapeDtypeStruct((NC, NPAD, D), jnp.float32),
        mesh=plsc.VectorSubcoreMesh(**_MESH),
        scratch_types=[
            pltpu.VMEM((CHUNK,), jnp.int32),
            pltpu.VMEM((CHUNK,), jnp.int32),
            pltpu.VMEM((CHUNK, D), jnp.float32),
            pltpu.VMEM((CHUNK,), jnp.int32),
            pltpu.VMEM((CHUNK,), jnp.int32),
            pltpu.VMEM((CHUNK, D), jnp.float32),
            pltpu.VMEM_SHARED((NPAD, D), jnp.float32),
            pltpu.SemaphoreType.DMA,
            pltpu.SemaphoreType.DMA,
        ],
    )
    return f(y, srcp, dstp, zeros)


def _dinv(deg_ref):
    dsum = deg_ref[0] + deg_ref[1]
    return lax.rsqrt(dsum[:, 0:1] + 1.0)


def _prep_body(deg_ref, x_ref, w_ref, t_ref, y_ref):
    dinv = _dinv(deg_ref)
    t = jnp.dot(x_ref[...], w_ref[...], preferred_element_type=jnp.float32)
    t_ref[...] = t
    y_ref[...] = dinv * t


def _mid_body(deg_ref, agg_ref, t_ref, b_ref, w_ref, tn_ref, yn_ref):
    dinv = _dinv(deg_ref)
    a = agg_ref[0] + agg_ref[1]
    h = jnp.maximum(dinv * a + (dinv * dinv) * t_ref[...] + b_ref[...], 0.0)
    t = jnp.dot(h, w_ref[...], preferred_element_type=jnp.float32)
    tn_ref[...] = t
    yn_ref[...] = dinv * t


def _final_body(deg_ref, agg_ref, t_ref, b_ref, out_ref):
    dinv = _dinv(deg_ref)
    a = agg_ref[0] + agg_ref[1]
    out_ref[...] = dinv * a + (dinv * dinv) * t_ref[...] + b_ref[...]


_DEG_SPEC = pl.BlockSpec((NC, ROWS, DEGW), lambda i: (0, i, 0))
_AGG_SPEC = pl.BlockSpec((NC, ROWS, D), lambda i: (0, i, 0))
_ROW_SPEC = pl.BlockSpec((ROWS, D), lambda i: (i, 0))
_W_SPEC = pl.BlockSpec((D, D), lambda i: (0, 0))
_B_SPEC = pl.BlockSpec((1, D), lambda i: (0, 0))
_ROW_OUT = jax.ShapeDtypeStruct((N, D), jnp.float32)


def _prep_call(deg, x, W0):
    return pl.pallas_call(
        _prep_body,
        grid=(GRID,),
        in_specs=[_DEG_SPEC, _ROW_SPEC, _W_SPEC],
        out_specs=[_ROW_SPEC, _ROW_SPEC],
        out_shape=[_ROW_OUT, _ROW_OUT],
    )(deg, x, W0)


def _mid_call(deg, agg, t, b, W):
    return pl.pallas_call(
        _mid_body,
        grid=(GRID,),
        in_specs=[_DEG_SPEC, _AGG_SPEC, _ROW_SPEC, _B_SPEC, _W_SPEC],
        out_specs=[_ROW_SPEC, _ROW_SPEC],
        out_shape=[_ROW_OUT, _ROW_OUT],
    )(deg, agg, t, b, W)


def _final_call(deg, agg, t, b):
    return pl.pallas_call(
        _final_body,
        grid=(GRID,),
        in_specs=[_DEG_SPEC, _AGG_SPEC, _ROW_SPEC, _B_SPEC],
        out_specs=_ROW_SPEC,
        out_shape=_ROW_OUT,
    )(deg, agg, t, b)


def kernel(x, edge_index, W0, b0, W1, b1, Wf, bf):
    src = edge_index[0].astype(jnp.int32)
    dst = edge_index[1].astype(jnp.int32)
    pad = EPAD - E
    pidx = jnp.arange(pad, dtype=jnp.int32)
    srcp = jnp.concatenate([src, pidx % N])
    dstp = jnp.concatenate([dst, N + pidx % (NPAD - N)])
    zeros = jnp.zeros((NPAD, D), jnp.float32)
    ones = jnp.ones((CHUNK, DEGW), jnp.float32)
    b0r = b0.reshape(1, D)
    b1r = b1.reshape(1, D)
    bfr = bf.reshape(1, D)

    deg = _deg_call(dstp, ones, zeros)
    t0, y0 = _prep_call(deg, x, W0)
    agg0 = _agg_call(y0, srcp, dstp, zeros)
    t1, y1 = _mid_call(deg, agg0, t0, b0r, W1)
    agg1 = _agg_call(y1, srcp, dstp, zeros)
    t2, y2 = _mid_call(deg, agg1, t1, b1r, Wf)
    agg2 = _agg_call(y2, srcp, dstp, zeros)
    return _final_call(deg, agg2, t2, bfr)


# speedup vs baseline: 2.6562x; 1.3059x over previous
import jax
import jax.numpy as jnp
from jax import lax
from jax.experimental import pallas as pl
from jax.experimental.pallas import tpu as pltpu
from jax.experimental.pallas import tpu_sc as plsc

N = 10000
D = 128
E = 320000
NC = 2
NS = 16
NW = NC * NS
CHUNK = 128
CPW = 80
EPAD = NW * CPW * CHUNK
ECH = EPAD // CHUNK
IBLK = 16
NPAD = 10112
RPT = NPAD // NS
XPT = NPAD // NS
DEGW = 128

ROWS = 2000
GRID = N // ROWS

_MESH = dict(core_axis_name="c", subcore_axis_name="s",
             num_cores=NC, num_subcores=NS)


def _deg_body(dst_hbm, ones_hbm, z_hbm, out_hbm, didx, onesv, acc, sem):
    c = lax.axis_index("c")
    s = lax.axis_index("s")
    w = s * NC + c
    pltpu.sync_copy(ones_hbm, onesv)
    pltpu.sync_copy(z_hbm.at[pl.ds(s * RPT, RPT)], acc.at[pl.ds(s * RPT, RPT)])
    plsc.subcore_barrier()

    def step(i, carry):
        off = (w * CPW + i) * CHUNK
        pltpu.sync_copy(dst_hbm.at[pl.ds(off, CHUNK)], didx)
        pltpu.sync_copy(onesv, acc.at[didx], add=True)
        return carry

    lax.fori_loop(0, CPW, step, 0)
    plsc.subcore_barrier()
    pltpu.sync_copy(acc.at[pl.ds(s * XPT, XPT)],
                    out_hbm.at[c, pl.ds(s * XPT, XPT)])


def _deg_call(dstp, ones, zeros):
    f = pl.kernel(
        _deg_body,
        out_type=jax.ShapeDtypeStruct((NC, NPAD, DEGW), jnp.float32),
        mesh=plsc.VectorSubcoreMesh(**_MESH),
        scratch_types=[
            pltpu.VMEM((CHUNK,), jnp.int32),
            pltpu.VMEM((CHUNK, DEGW), jnp.float32),
            pltpu.VMEM_SHARED((NPAD, DEGW), jnp.float32),
            pltpu.SemaphoreType.DMA,
        ],
    )
    return f(dstp, ones, zeros)


def _agg_body(y_hbm, src_hbm, dst_hbm, z_hbm, out_hbm,
              sidx_a, didx_a, rows_a, sidx_b, didx_b, rows_b, acc,
              sem_a, sem_b):
    c = lax.axis_index("c")
    s = lax.axis_index("s")
    w = s * NC + c
    base = w * CPW * CHUNK
    pltpu.sync_copy(z_hbm.at[pl.ds(s * RPT, RPT)], acc.at[pl.ds(s * RPT, RPT)])
    plsc.subcore_barrier()

    def step(i, carry):
        off_a = base + (2 * i) * CHUNK
        off_b = off_a + CHUNK
        pltpu.sync_copy(src_hbm.at[pl.ds(off_a, CHUNK)], sidx_a)
        pltpu.sync_copy(dst_hbm.at[pl.ds(off_a, CHUNK)], didx_a)
        ga = pltpu.async_copy(y_hbm.at[sidx_a], rows_a, sem_a)
        pltpu.sync_copy(src_hbm.at[pl.ds(off_b, CHUNK)], sidx_b)
        pltpu.sync_copy(dst_hbm.at[pl.ds(off_b, CHUNK)], didx_b)
        gb = pltpu.async_copy(y_hbm.at[sidx_b], rows_b, sem_b)
        ga.wait()
        pltpu.sync_copy(rows_a, acc.at[didx_a], add=True)
        gb.wait()
        pltpu.sync_copy(rows_b, acc.at[didx_b], add=True)
        return carry

    lax.fori_loop(0, CPW // 2, step, 0)
    plsc.subcore_barrier()
    pltpu.sync_copy(acc.at[pl.ds(s * XPT, XPT)],
                    out_hbm.at[c, pl.ds(s * XPT, XPT)])


def _agg_call(y, srcp, dstp, zeros):
    f = pl.kernel(
        _agg_body,
        out_type=jax.ShapeDtypeStruct((NC, NPAD, D), jnp.float32),
        mesh=plsc.VectorSubcoreMesh(**_MESH),
        scratch_types=[
            pltpu.VMEM((CHUNK,), jnp.int32),
            pltpu.VMEM((CHUNK,), jnp.int32),
            pltpu.VMEM((CHUNK, D), jnp.float32),
            pltpu.VMEM((CHUNK,), jnp.int32),
            pltpu.VMEM((CHUNK,), jnp.int32),
            pltpu.VMEM((CHUNK, D), jnp.float32),
            pltpu.VMEM_SHARED((NPAD, D), jnp.float32),
            pltpu.SemaphoreType.DMA,
            pltpu.SemaphoreType.DMA,
        ],
    )
    return f(y, srcp, dstp, zeros)


def _dinv(deg_ref):
    dsum = deg_ref[0] + deg_ref[1]
    return lax.rsqrt(dsum[:, 0:1] + 1.0)


def _prep_body(deg_ref, x_ref, w_ref, t_ref, y_ref):
    dinv = _dinv(deg_ref)
    t = jnp.dot(x_ref[...], w_ref[...], preferred_element_type=jnp.float32)
    t_ref[...] = t
    y_ref[...] = dinv * t


def _mid_body(deg_ref, agg_ref, t_ref, b_ref, w_ref, tn_ref, yn_ref):
    dinv = _dinv(deg_ref)
    a = agg_ref[0] + agg_ref[1]
    h = jnp.maximum(dinv * a + (dinv * dinv) * t_ref[...] + b_ref[...], 0.0)
    t = jnp.dot(h, w_ref[...], preferred_element_type=jnp.float32)
    tn_ref[...] = t
    yn_ref[...] = dinv * t


def _final_body(deg_ref, agg_ref, t_ref, b_ref, out_ref):
    dinv = _dinv(deg_ref)
    a = agg_ref[0] + agg_ref[1]
    out_ref[...] = dinv * a + (dinv * dinv) * t_ref[...] + b_ref[...]


_DEG_SPEC = pl.BlockSpec((NC, ROWS, DEGW), lambda i: (0, i, 0))
_AGG_SPEC = pl.BlockSpec((NC, ROWS, D), lambda i: (0, i, 0))
_ROW_SPEC = pl.BlockSpec((ROWS, D), lambda i: (i, 0))
_W_SPEC = pl.BlockSpec((D, D), lambda i: (0, 0))
_B_SPEC = pl.BlockSpec((1, D), lambda i: (0, 0))
_ROW_OUT = jax.ShapeDtypeStruct((N, D), jnp.float32)


def _prep_call(deg, x, W0):
    return pl.pallas_call(
        _prep_body,
        grid=(GRID,),
        in_specs=[_DEG_SPEC, _ROW_SPEC, _W_SPEC],
        out_specs=[_ROW_SPEC, _ROW_SPEC],
        out_shape=[_ROW_OUT, _ROW_OUT],
    )(deg, x, W0)


def _mid_call(deg, agg, t, b, W):
    return pl.pallas_call(
        _mid_body,
        grid=(GRID,),
        in_specs=[_DEG_SPEC, _AGG_SPEC, _ROW_SPEC, _B_SPEC, _W_SPEC],
        out_specs=[_ROW_SPEC, _ROW_SPEC],
        out_shape=[_ROW_OUT, _ROW_OUT],
    )(deg, agg, t, b, W)


def _final_call(deg, agg, t, b):
    return pl.pallas_call(
        _final_body,
        grid=(GRID,),
        in_specs=[_DEG_SPEC, _AGG_SPEC, _ROW_SPEC, _B_SPEC],
        out_specs=_ROW_SPEC,
        out_shape=_ROW_OUT,
    )(deg, agg, t, b)


def kernel(x, edge_index, W0, b0, W1, b1, Wf, bf):
    src = edge_index[0].astype(jnp.int32)
    dst = edge_index[1].astype(jnp.int32)
    pad = EPAD - E
    pidx = jnp.arange(pad, dtype=jnp.int32)
    srcp = jnp.concatenate([src, pidx % N])
    dstp = jnp.concatenate([dst, N + pidx % (NPAD - N)])
    zeros = jnp.zeros((NPAD, D), jnp.float32)
    ones = jnp.ones((CHUNK, DEGW), jnp.float32)
    b0r = b0.reshape(1, D)
    b1r = b1.reshape(1, D)
    bfr = bf.reshape(1, D)

    deg = _deg_call(dstp, ones, zeros)
    t0, y0 = _prep_call(deg, x, W0)
    agg0 = _agg_call(y0, srcp, dstp, zeros)
    t1, y1 = _mid_call(deg, agg0, t0, b0r, W1)
    agg1 = _agg_call(y1, srcp, dstp, zeros)
    t2, y2 = _mid_call(deg, agg1, t1, b1r, Wf)
    agg2 = _agg_call(y2, srcp, dstp, zeros)
    return _final_call(deg, agg2, t2, bfr)


# speedup vs baseline: 2.6576x; 1.0005x over previous
import jax
import jax.numpy as jnp
from jax import lax
from jax.experimental import pallas as pl
from jax.experimental.pallas import tpu as pltpu
from jax.experimental.pallas import tpu_sc as plsc

N = 10000
D = 128
E = 320000
NC = 2
NS = 16
NW = NC * NS
CHUNK = 128
CPW = 80
EPAD = NW * CPW * CHUNK
ECH = EPAD // CHUNK
IBLK = 16
NPAD = 10112
RPT = NPAD // NS
XPT = NPAD // NS
DEGW = 128

ROWS = 2000
GRID = N // ROWS

_MESH = dict(core_axis_name="c", subcore_axis_name="s",
             num_cores=NC, num_subcores=NS)


def _deg_body(dst_hbm, ones_hbm, z_hbm, out_hbm, didx, onesv, acc, sem):
    c = lax.axis_index("c")
    s = lax.axis_index("s")
    w = s * NC + c
    pltpu.sync_copy(ones_hbm, onesv)
    pltpu.sync_copy(z_hbm.at[pl.ds(s * RPT, RPT)], acc.at[pl.ds(s * RPT, RPT)])
    plsc.subcore_barrier()

    def step(i, carry):
        off = (w * CPW + i) * CHUNK
        pltpu.sync_copy(dst_hbm.at[pl.ds(off, CHUNK)], didx)
        pltpu.sync_copy(onesv, acc.at[didx], add=True)
        return carry

    lax.fori_loop(0, CPW, step, 0)
    plsc.subcore_barrier()
    pltpu.sync_copy(acc.at[pl.ds(s * XPT, XPT)],
                    out_hbm.at[c, pl.ds(s * XPT, XPT)])


def _deg_call(dstp, ones, zeros):
    f = pl.kernel(
        _deg_body,
        out_type=jax.ShapeDtypeStruct((NC, NPAD, DEGW), jnp.float32),
        mesh=plsc.VectorSubcoreMesh(**_MESH),
        scratch_types=[
            pltpu.VMEM((CHUNK,), jnp.int32),
            pltpu.VMEM((CHUNK, DEGW), jnp.float32),
            pltpu.VMEM_SHARED((NPAD, DEGW), jnp.float32),
            pltpu.SemaphoreType.DMA,
        ],
    )
    return f(dstp, ones, zeros)


def _agg_body(y_hbm, src_hbm, dst_hbm, z_hbm, out_hbm,
              sidx_a, didx_a, rows_a, sidx_b, didx_b, rows_b, acc,
              sem_a, sem_b, ssem_a, ssem_b):
    c = lax.axis_index("c")
    s = lax.axis_index("s")
    w = s * NC + c
    base = w * CPW * CHUNK
    pltpu.sync_copy(z_hbm.at[pl.ds(s * RPT, RPT)], acc.at[pl.ds(s * RPT, RPT)])
    plsc.subcore_barrier()

    def step(i, carry):
        off_a = base + (2 * i) * CHUNK
        off_b = off_a + CHUNK
        pltpu.sync_copy(src_hbm.at[pl.ds(off_a, CHUNK)], sidx_a)
        pltpu.sync_copy(dst_hbm.at[pl.ds(off_a, CHUNK)], didx_a)
        ga = pltpu.async_copy(y_hbm.at[sidx_a], rows_a, sem_a)
        pltpu.sync_copy(src_hbm.at[pl.ds(off_b, CHUNK)], sidx_b)
        pltpu.sync_copy(dst_hbm.at[pl.ds(off_b, CHUNK)], didx_b)
        gb = pltpu.async_copy(y_hbm.at[sidx_b], rows_b, sem_b)
        ga.wait()
        sa = pltpu.async_copy(rows_a, acc.at[didx_a], ssem_a, add=True)
        gb.wait()
        sb = pltpu.async_copy(rows_b, acc.at[didx_b], ssem_b, add=True)
        sa.wait()
        sb.wait()
        return carry

    lax.fori_loop(0, CPW // 2, step, 0)
    plsc.subcore_barrier()
    pltpu.sync_copy(acc.at[pl.ds(s * XPT, XPT)],
                    out_hbm.at[c, pl.ds(s * XPT, XPT)])


def _agg_call(y, srcp, dstp, zeros):
    f = pl.kernel(
        _agg_body,
        out_type=jax.ShapeDtypeStruct((NC, NPAD, D), jnp.float32),
        mesh=plsc.VectorSubcoreMesh(**_MESH),
        scratch_types=[
            pltpu.VMEM((CHUNK,), jnp.int32),
            pltpu.VMEM((CHUNK,), jnp.int32),
            pltpu.VMEM((CHUNK, D), jnp.float32),
            pltpu.VMEM((CHUNK,), jnp.int32),
            pltpu.VMEM((CHUNK,), jnp.int32),
            pltpu.VMEM((CHUNK, D), jnp.float32),
            pltpu.VMEM_SHARED((NPAD, D), jnp.float32),
            pltpu.SemaphoreType.DMA,
            pltpu.SemaphoreType.DMA,
            pltpu.SemaphoreType.DMA,
            pltpu.SemaphoreType.DMA,
        ],
    )
    return f(y, srcp, dstp, zeros)


def _dinv(deg_ref):
    dsum = deg_ref[0] + deg_ref[1]
    return lax.rsqrt(dsum[:, 0:1] + 1.0)


def _prep_body(deg_ref, x_ref, w_ref, t_ref, y_ref):
    dinv = _dinv(deg_ref)
    t = jnp.dot(x_ref[...], w_ref[...], preferred_element_type=jnp.float32)
    t_ref[...] = t
    y_ref[...] = dinv * t


def _mid_body(deg_ref, agg_ref, t_ref, b_ref, w_ref, tn_ref, yn_ref):
    dinv = _dinv(deg_ref)
    a = agg_ref[0] + agg_ref[1]
    h = jnp.maximum(dinv * a + (dinv * dinv) * t_ref[...] + b_ref[...], 0.0)
    t = jnp.dot(h, w_ref[...], preferred_element_type=jnp.float32)
    tn_ref[...] = t
    yn_ref[...] = dinv * t


def _final_body(deg_ref, agg_ref, t_ref, b_ref, out_ref):
    dinv = _dinv(deg_ref)
    a = agg_ref[0] + agg_ref[1]
    out_ref[...] = dinv * a + (dinv * dinv) * t_ref[...] + b_ref[...]


_DEG_SPEC = pl.BlockSpec((NC, ROWS, DEGW), lambda i: (0, i, 0))
_AGG_SPEC = pl.BlockSpec((NC, ROWS, D), lambda i: (0, i, 0))
_ROW_SPEC = pl.BlockSpec((ROWS, D), lambda i: (i, 0))
_W_SPEC = pl.BlockSpec((D, D), lambda i: (0, 0))
_B_SPEC = pl.BlockSpec((1, D), lambda i: (0, 0))
_ROW_OUT = jax.ShapeDtypeStruct((N, D), jnp.float32)


def _prep_call(deg, x, W0):
    return pl.pallas_call(
        _prep_body,
        grid=(GRID,),
        in_specs=[_DEG_SPEC, _ROW_SPEC, _W_SPEC],
        out_specs=[_ROW_SPEC, _ROW_SPEC],
        out_shape=[_ROW_OUT, _ROW_OUT],
    )(deg, x, W0)


def _mid_call(deg, agg, t, b, W):
    return pl.pallas_call(
        _mid_body,
        grid=(GRID,),
        in_specs=[_DEG_SPEC, _AGG_SPEC, _ROW_SPEC, _B_SPEC, _W_SPEC],
        out_specs=[_ROW_SPEC, _ROW_SPEC],
        out_shape=[_ROW_OUT, _ROW_OUT],
    )(deg, agg, t, b, W)


def _final_call(deg, agg, t, b):
    return pl.pallas_call(
        _final_body,
        grid=(GRID,),
        in_specs=[_DEG_SPEC, _AGG_SPEC, _ROW_SPEC, _B_SPEC],
        out_specs=_ROW_SPEC,
        out_shape=_ROW_OUT,
    )(deg, agg, t, b)


def kernel(x, edge_index, W0, b0, W1, b1, Wf, bf):
    src = edge_index[0].astype(jnp.int32)
    dst = edge_index[1].astype(jnp.int32)
    pad = EPAD - E
    pidx = jnp.arange(pad, dtype=jnp.int32)
    srcp = jnp.concatenate([src, pidx % N])
    dstp = jnp.concatenate([dst, N + pidx % (NPAD - N)])
    zeros = jnp.zeros((NPAD, D), jnp.float32)
    ones = jnp.ones((CHUNK, DEGW), jnp.float32)
    b0r = b0.reshape(1, D)
    b1r = b1.reshape(1, D)
    bfr = bf.reshape(1, D)

    deg = _deg_call(dstp, ones, zeros)
    t0, y0 = _prep_call(deg, x, W0)
    agg0 = _agg_call(y0, srcp, dstp, zeros)
    t1, y1 = _mid_call(deg, agg0, t0, b0r, W1)
    agg1 = _agg_call(y1, srcp, dstp, zeros)
    t2, y2 = _mid_call(deg, agg1, t1, b1r, Wf)
    agg2 = _agg_call(y2, srcp, dstp, zeros)
    return _final_call(deg, agg2, t2, bfr)
